# baseline (speedup 1.0000x reference)
import jax
import jax.numpy as jnp
from jax.experimental import pallas as pl

K = 20
EPS = 1e-5


def lrelu(x):
    return jax.nn.leaky_relu(x, negative_slope=0.2)


def _knn(x, k):
    inner = -2.0 * jnp.einsum('bcn,bcm->bnm', x, x)
    xx = jnp.sum(x * x, axis=1, keepdims=True)
    pd = -xx - inner - jnp.transpose(xx, (0, 2, 1))
    return jax.lax.top_k(pd, k)[1]


def _get_graph_feature(x, k):
    b, c, n = x.shape
    idx = _knn(x, k)
    xt = jnp.transpose(x, (0, 2, 1))
    feat = xt[jnp.arange(b)[:, None, None], idx]
    xc = jnp.broadcast_to(xt[:, :, None, :], (b, n, k, c))
    feat = jnp.concatenate([feat - xc, xc], axis=3)
    feat = jnp.transpose(feat, (0, 3, 1, 2))
    idx_g = (idx + (jnp.arange(b) * n)[:, None, None]).reshape(-1)
    return feat, idx_g


def _bn(x, axes):
    m = jnp.mean(x, axis=axes, keepdims=True)
    v = jnp.var(x, axis=axes, keepdims=True)
    return (x - m) / jnp.sqrt(v + EPS)


def _channel_attention(x, w1, w2):
    avg_in = jnp.mean(x, axis=(2, 3), keepdims=True)
    max_in = jnp.max(x, axis=(2, 3), keepdims=True)
    def fc(t):
        h = lrelu(jnp.einsum('bcij,oc->boij', t, w1))
        return jnp.einsum('bcij,oc->boij', h, w2)
    return jax.nn.sigmoid(fc(avg_in) + fc(max_in))


def _spatial_attention(x, w):
    avg = jnp.mean(x, axis=1, keepdims=True)
    mx = jnp.max(x, axis=1, keepdims=True)
    cat = jnp.concatenate([avg, mx], axis=1)
    return jax.nn.sigmoid(jnp.einsum('bchw,oc->bohw', cat, w))


def _edge_block(x, wconv, caw1, caw2, saw):
    feat, idx_g = _get_graph_feature(x, K)
    x11 = jnp.einsum('bcnk,oc->bonk', feat, wconv)
    x12 = x11[..., :10]
    x13 = x11[..., :5]
    i11 = jnp.argmax(x11, axis=-1)
    i12 = jnp.argmax(x12, axis=-1)
    i13 = jnp.argmax(x13, axis=-1)
    s = jnp.stack([jnp.max(x11, axis=-1), jnp.max(x12, axis=-1), jnp.max(x13, axis=-1)], axis=3)
    s = lrelu(_bn(s, (0, 2, 3)))
    s = _channel_attention(s, caw1, caw2) * s
    sa = _spatial_attention(s, saw)
    out = jnp.mean(sa * s, axis=-1)
    return out, idx_g, (i11, i12, i13), sa


def _copy_kernel(x_ref, o_ref):
    o_ref[...] = x_ref[...]


def kernel(x, W1, ca1a, ca1b, sa1w, W2, ca2a, ca2b, sa2w, W3, ca3a, ca3b, sa3w, W4, ca4a, ca4b, sa4w, W5, L1, L2, b2, L3, b3):
    x1, _, _, _ = _edge_block(x, W1, ca1a, ca1b, sa1w)
    x2, _, _, _ = _edge_block(x1, W2, ca2a, ca2b, sa2w)
    x3, _, _, _ = _edge_block(x2, W3, ca3a, ca3b, sa3w)
    x4, idx_g, inds, sa4 = _edge_block(x3, W4, ca4a, ca4b, sa4w)
    idx = idx_g.reshape(-1, K)
    x5idx = jnp.argmax(sa4, axis=-1)[0][0]
    xc = jnp.concatenate([x1, x2, x3, x4], axis=1)
    h = jnp.einsum('bcn,oc->bon', xc, W5)
    h = lrelu(_bn(h, (0, 2)))
    max_vals = jnp.max(h, axis=2)
    indices = jnp.argmax(h, axis=2)
    avg_vals = jnp.mean(h, axis=2)
    hh = jnp.concatenate([max_vals, avg_vals], axis=1)
    hh = lrelu(_bn(hh @ L1.T, (0,)))
    hh = lrelu(_bn(hh @ L2.T + b2, (0,)))
    out = hh @ L3.T + b3
    out = pl.pallas_call(
        _copy_kernel,
        out_shape=jax.ShapeDtypeStruct(out.shape, out.dtype),
    )(out)
    return out, indices, inds, idx, x5idx


# speedup vs baseline: 6.2044x; 6.2044x over previous
import functools

import jax
import jax.numpy as jnp
from jax import lax
from jax.experimental import pallas as pl
from jax.experimental.pallas import tpu as pltpu

K = 20
EPS = 1e-5
B, N = 8, 1024
NEG = float("-inf")


def _lrelu(x):
    return jax.nn.leaky_relu(x, negative_slope=0.2)


def _bn(x, axes):
    m = jnp.mean(x, axis=axes, keepdims=True)
    v = jnp.var(x, axis=axes, keepdims=True)
    return (x - m) / jnp.sqrt(v + EPS)


def _channel_attention(x, w1, w2):
    avg_in = jnp.mean(x, axis=(2, 3), keepdims=True)
    max_in = jnp.max(x, axis=(2, 3), keepdims=True)
    def fc(t):
        h = _lrelu(jnp.einsum('bcij,oc->boij', t, w1))
        return jnp.einsum('bcij,oc->boij', h, w2)
    return jax.nn.sigmoid(fc(avg_in) + fc(max_in))


def _spatial_attention(x, w):
    avg = jnp.mean(x, axis=1, keepdims=True)
    mx = jnp.max(x, axis=1, keepdims=True)
    cat = jnp.concatenate([avg, mx], axis=1)
    return jax.nn.sigmoid(jnp.einsum('bchw,oc->bohw', cat, w))


def _edge_body(x_ref, xx_ref, w_ref, x11_ref, idx_ref, pd_ref, *, with_idx):
    n = N
    j = pl.program_id(1)
    x = x_ref[0]
    xx = xx_ref[0]

    @pl.when(j == 0)
    def _():
        dots = lax.dot_general(x, x, (((0,), (0,)), ((), ())),
                               preferred_element_type=jnp.float32)
        inner = -2.0 * dots
        xx_col = jnp.reshape(xx, (n, 1))
        pd_ref[...] = (-xx_col - inner) - xx

    rowf = lax.broadcasted_iota(jnp.int32, (n, n), 0).astype(jnp.float32)

    pd = pd_ref[...]
    v = jnp.max(pd, axis=0, keepdims=True)
    af = jnp.min(jnp.where(pd == v, rowf, float(n)), axis=0,
                 keepdims=True)
    hot = rowf == af
    pd_ref[...] = jnp.where(hot, NEG, pd)
    g = lax.dot_general(x, hot.astype(jnp.float32),
                        (((1,), (0,)), ((), ())),
                        precision=lax.Precision.HIGHEST,
                        preferred_element_type=jnp.float32)
    feat = jnp.concatenate([g - x, x], axis=0)
    w = w_ref[...]
    val = lax.dot_general(w, feat, (((1,), (0,)), ((), ())),
                          preferred_element_type=jnp.float32)
    x11_ref[0, 0] = val
    if with_idx:
        idx_ref[0, 0] = af.astype(jnp.int32)


def _edge_pallas(x, xx, w, *, with_idx):
    b, c, n = x.shape
    o = w.shape[0]
    outs = [jax.ShapeDtypeStruct((b, K, o, n), jnp.float32)]
    out_specs = [pl.BlockSpec((1, 1, o, n), lambda i, j: (i, j, 0, 0))]
    if with_idx:
        outs.append(jax.ShapeDtypeStruct((b, K, 1, n), jnp.int32))
        out_specs.append(pl.BlockSpec((1, 1, 1, n), lambda i, j: (i, j, 0, 0)))
    else:
        outs.append(jax.ShapeDtypeStruct((1, 1, 1, n), jnp.int32))
        out_specs.append(pl.BlockSpec((1, 1, 1, n), lambda i, j: (0, 0, 0, 0)))

    body = functools.partial(_edge_body, with_idx=with_idx)
    return pl.pallas_call(
        body,
        grid=(b, K),
        in_specs=[
            pl.BlockSpec((1, c, n), lambda i, j: (i, 0, 0)),
            pl.BlockSpec((1, 1, n), lambda i, j: (i, 0, 0)),
            pl.BlockSpec(w.shape, lambda i, j: (0, 0)),
        ],
        out_specs=out_specs,
        out_shape=outs,
        scratch_shapes=[pltpu.VMEM((n, n), jnp.float32)],
    )(x, xx, w)


def _edge_block_fast(x, wconv, caw1, caw2, saw, *, with_idx):
    xx = jnp.sum(x * x, axis=1, keepdims=True)
    x11_out, idx_out = _edge_pallas(x, xx, wconv, with_idx=with_idx)
    x11 = jnp.transpose(x11_out, (0, 2, 3, 1))
    x12 = x11[..., :10]
    x13 = x11[..., :5]
    s = jnp.stack([jnp.max(x11, axis=-1), jnp.max(x12, axis=-1),
                   jnp.max(x13, axis=-1)], axis=3)
    s = _lrelu(_bn(s, (0, 2, 3)))
    s = _channel_attention(s, caw1, caw2) * s
    sa = _spatial_attention(s, saw)
    out = jnp.mean(sa * s, axis=-1)
    if with_idx:
        i11 = jnp.argmax(x11, axis=-1)
        i12 = jnp.argmax(x12, axis=-1)
        i13 = jnp.argmax(x13, axis=-1)
        idx = jnp.transpose(idx_out[:, :, 0, :], (0, 2, 1))
        return out, idx, (i11, i12, i13), sa
    return out, None, None, sa


def _head_body(w_ref, xc_ref, h_ref):
    h_ref[0] = lax.dot_general(w_ref[...], xc_ref[0],
                               (((1,), (0,)), ((), ())),
                               preferred_element_type=jnp.float32)


def _head_pallas(w5, xc):
    b, c, n = xc.shape
    o = w5.shape[0]
    return pl.pallas_call(
        _head_body,
        grid=(b,),
        in_specs=[
            pl.BlockSpec(w5.shape, lambda i: (0, 0)),
            pl.BlockSpec((1, c, n), lambda i: (i, 0, 0)),
        ],
        out_specs=pl.BlockSpec((1, o, n), lambda i: (i, 0, 0)),
        out_shape=jax.ShapeDtypeStruct((b, o, n), jnp.float32),
    )(w5, xc)


def kernel(x, W1, ca1a, ca1b, sa1w, W2, ca2a, ca2b, sa2w, W3, ca3a, ca3b, sa3w, W4, ca4a, ca4b, sa4w, W5, L1, L2, b2, L3, b3):
    x1, _, _, _ = _edge_block_fast(x, W1, ca1a, ca1b, sa1w, with_idx=False)
    x2, _, _, _ = _edge_block_fast(x1, W2, ca2a, ca2b, sa2w, with_idx=False)
    x3, _, _, _ = _edge_block_fast(x2, W3, ca3a, ca3b, sa3w, with_idx=False)
    x4, idx_l, inds, sa4 = _edge_block_fast(x3, W4, ca4a, ca4b, sa4w, with_idx=True)

    idx = (idx_l + (jnp.arange(B) * N)[:, None, None]).reshape(-1, K)
    x5idx = jnp.argmax(sa4, axis=-1)[0][0]

    xc = jnp.concatenate([x1, x2, x3, x4], axis=1)
    h = _head_pallas(W5, xc)
    h = _lrelu(_bn(h, (0, 2)))
    max_vals = jnp.max(h, axis=2)
    indices = jnp.argmax(h, axis=2)
    avg_vals = jnp.mean(h, axis=2)
    hh = jnp.concatenate([max_vals, avg_vals], axis=1)
    hh = _lrelu(_bn(hh @ L1.T, (0,)))
    hh = _lrelu(_bn(hh @ L2.T + b2, (0,)))
    out = hh @ L3.T + b3
    return out, indices, inds, idx, x5idx


# speedup vs baseline: 6.6179x; 1.0666x over previous
import functools

import jax
import jax.numpy as jnp
from jax import lax
from jax.experimental import pallas as pl
from jax.experimental.pallas import tpu as pltpu

K = 20
EPS = 1e-5
B, N = 8, 1024
NEG = float("-inf")


def _lrelu(x):
    return jax.nn.leaky_relu(x, negative_slope=0.2)


def _bn(x, axes):
    m = jnp.mean(x, axis=axes, keepdims=True)
    v = jnp.var(x, axis=axes, keepdims=True)
    return (x - m) / jnp.sqrt(v + EPS)


def _channel_attention(x, w1, w2):
    avg_in = jnp.mean(x, axis=(2, 3), keepdims=True)
    max_in = jnp.max(x, axis=(2, 3), keepdims=True)
    def fc(t):
        h = _lrelu(jnp.einsum('bcij,oc->boij', t, w1))
        return jnp.einsum('bcij,oc->boij', h, w2)
    return jax.nn.sigmoid(fc(avg_in) + fc(max_in))


def _spatial_attention(x, w):
    avg = jnp.mean(x, axis=1, keepdims=True)
    mx = jnp.max(x, axis=1, keepdims=True)
    cat = jnp.concatenate([avg, mx], axis=1)
    return jax.nn.sigmoid(jnp.einsum('bchw,oc->bohw', cat, w))


UNROLL = 2


def _edge_body(x_ref, xx_ref, w_ref, x11_ref, idx_ref, pd_ref, *, with_idx):
    n = N
    jj = pl.program_id(1)
    x = x_ref[0]
    xx = xx_ref[0]

    @pl.when(jj == 0)
    def _():
        dots = lax.dot_general(x, x, (((0,), (0,)), ((), ())),
                               preferred_element_type=jnp.float32)
        inner = -2.0 * dots
        xx_col = jnp.reshape(xx, (n, 1))
        pd_ref[...] = (-xx_col - inner) - xx

    rowf = lax.broadcasted_iota(jnp.int32, (n, n), 0).astype(jnp.float32)

    for u in range(UNROLL):
        pd = pd_ref[...]
        v = jnp.max(pd, axis=0, keepdims=True)
        af = jnp.min(jnp.where(pd == v, rowf, float(n)), axis=0,
                     keepdims=True)
        hot = rowf == af
        pd_ref[...] = jnp.where(hot, NEG, pd)
        g = lax.dot_general(x, hot.astype(jnp.float32),
                            (((1,), (0,)), ((), ())),
                            precision=lax.Precision.HIGHEST,
                            preferred_element_type=jnp.float32)
        feat = jnp.concatenate([g - x, x], axis=0)
        w = w_ref[...]
        val = lax.dot_general(w, feat, (((1,), (0,)), ((), ())),
                              preferred_element_type=jnp.float32)
        x11_ref[0, u] = val
        if with_idx:
            idx_ref[0, u] = af.astype(jnp.int32)


def _edge_pallas(x, xx, w, *, with_idx):
    b, c, n = x.shape
    o = w.shape[0]
    outs = [jax.ShapeDtypeStruct((b, K, o, n), jnp.float32)]
    out_specs = [pl.BlockSpec((1, UNROLL, o, n), lambda i, j: (i, j, 0, 0))]
    if with_idx:
        outs.append(jax.ShapeDtypeStruct((b, K, 1, n), jnp.int32))
        out_specs.append(pl.BlockSpec((1, UNROLL, 1, n), lambda i, j: (i, j, 0, 0)))
    else:
        outs.append(jax.ShapeDtypeStruct((1, UNROLL, 1, n), jnp.int32))
        out_specs.append(pl.BlockSpec((1, UNROLL, 1, n), lambda i, j: (0, 0, 0, 0)))

    body = functools.partial(_edge_body, with_idx=with_idx)
    return pl.pallas_call(
        body,
        grid=(b, K // UNROLL),
        in_specs=[
            pl.BlockSpec((1, c, n), lambda i, j: (i, 0, 0)),
            pl.BlockSpec((1, 1, n), lambda i, j: (i, 0, 0)),
            pl.BlockSpec(w.shape, lambda i, j: (0, 0)),
        ],
        out_specs=out_specs,
        out_shape=outs,
        scratch_shapes=[pltpu.VMEM((n, n), jnp.float32)],
    )(x, xx, w)


def _edge_block_fast(x, wconv, caw1, caw2, saw, *, with_idx):
    xx = jnp.sum(x * x, axis=1, keepdims=True)
    x11_out, idx_out = _edge_pallas(x, xx, wconv, with_idx=with_idx)
    x11 = jnp.transpose(x11_out, (0, 2, 3, 1))
    x12 = x11[..., :10]
    x13 = x11[..., :5]
    s = jnp.stack([jnp.max(x11, axis=-1), jnp.max(x12, axis=-1),
                   jnp.max(x13, axis=-1)], axis=3)
    s = _lrelu(_bn(s, (0, 2, 3)))
    s = _channel_attention(s, caw1, caw2) * s
    sa = _spatial_attention(s, saw)
    out = jnp.mean(sa * s, axis=-1)
    if with_idx:
        i11 = jnp.argmax(x11, axis=-1)
        i12 = jnp.argmax(x12, axis=-1)
        i13 = jnp.argmax(x13, axis=-1)
        idx = jnp.transpose(idx_out[:, :, 0, :], (0, 2, 1))
        return out, idx, (i11, i12, i13), sa
    return out, None, None, sa


def _head_body(w_ref, xc_ref, h_ref):
    h_ref[0] = lax.dot_general(w_ref[...], xc_ref[0],
                               (((1,), (0,)), ((), ())),
                               preferred_element_type=jnp.float32)


def _head_pallas(w5, xc):
    b, c, n = xc.shape
    o = w5.shape[0]
    return pl.pallas_call(
        _head_body,
        grid=(b,),
        in_specs=[
            pl.BlockSpec(w5.shape, lambda i: (0, 0)),
            pl.BlockSpec((1, c, n), lambda i: (i, 0, 0)),
        ],
        out_specs=pl.BlockSpec((1, o, n), lambda i: (i, 0, 0)),
        out_shape=jax.ShapeDtypeStruct((b, o, n), jnp.float32),
    )(w5, xc)


def kernel(x, W1, ca1a, ca1b, sa1w, W2, ca2a, ca2b, sa2w, W3, ca3a, ca3b, sa3w, W4, ca4a, ca4b, sa4w, W5, L1, L2, b2, L3, b3):
    x1, _, _, _ = _edge_block_fast(x, W1, ca1a, ca1b, sa1w, with_idx=False)
    x2, _, _, _ = _edge_block_fast(x1, W2, ca2a, ca2b, sa2w, with_idx=False)
    x3, _, _, _ = _edge_block_fast(x2, W3, ca3a, ca3b, sa3w, with_idx=False)
    x4, idx_l, inds, sa4 = _edge_block_fast(x3, W4, ca4a, ca4b, sa4w, with_idx=True)

    idx = (idx_l + (jnp.arange(B) * N)[:, None, None]).reshape(-1, K)
    x5idx = jnp.argmax(sa4, axis=-1)[0][0]

    xc = jnp.concatenate([x1, x2, x3, x4], axis=1)
    h = _head_pallas(W5, xc)
    h = _lrelu(_bn(h, (0, 2)))
    max_vals = jnp.max(h, axis=2)
    indices = jnp.argmax(h, axis=2)
    avg_vals = jnp.mean(h, axis=2)
    hh = jnp.concatenate([max_vals, avg_vals], axis=1)
    hh = _lrelu(_bn(hh @ L1.T, (0,)))
    hh = _lrelu(_bn(hh @ L2.T + b2, (0,)))
    out = hh @ L3.T + b3
    return out, indices, inds, idx, x5idx
